# baseline (speedup 1.0000x reference)
import functools

import jax
import jax.numpy as jnp
from jax import lax
from jax.experimental import pallas as pl
from jax.experimental.pallas import tpu as pltpu
from jax.experimental.pallas import tpu_sc as plsc

_H = 512
_W = 512
_C = 32
_PW = _W + 2
_PH = 520
_R = 8
_NS = 9
_NH = 2
_DH = 16
_SHIFTS = [[0, 0], [-1, 0], [1, 0], [0, 1], [-1, 1], [1, 1], [0, -1], [-1, -1], [1, -1]]


def _attn_body(pos2_ref, bink_ref, binv_ref, outwT_ref, outb_ref,
               xa_ref, xb_ref, ya_ref, yb_ref, out_ref):
    X = jnp.concatenate([xa_ref[0], xb_ref[0]], axis=0)
    Y = jnp.concatenate([ya_ref[0], yb_ref[0]], axis=0)
    M = _R * _W
    q = X[1:_R + 1, 1:_W + 1, 0:_C].reshape(M, _C) * (1.0 / (_DH ** 0.5))
    mx = X[1:_R + 1, 1:_W + 1, 96:97].reshape(M, 1)
    bink = bink_ref[0]
    binv = binv_ref[0]
    prods = []
    vals = []
    for j, (dy, dx) in enumerate(_SHIFTS):
        Ys = Y[1 + dy:1 + dy + _R, 1 + dx:1 + dx + _W, :]
        kj = Ys[:, :, _C:2 * _C].reshape(M, _C)
        vj = Ys[:, :, 2 * _C:3 * _C].reshape(M, _C)
        mj = Ys[:, :, 96:97].reshape(M, 1)
        ok = mj > 0.0
        prods.append(q * jnp.where(ok, kj, bink))
        vals.append(jnp.where(ok, vj + pos2_ref[0, j, :][None, :], binv))
    pc = jnp.concatenate(prods, axis=1)
    vc = jnp.concatenate(vals, axis=1)
    r_i = jax.lax.broadcasted_iota(jnp.int32, (_NS * _C, _NS * _NH), 0)
    c_i = jax.lax.broadcasted_iota(jnp.int32, (_NS * _C, _NS * _NH), 1)
    S = (c_i == _NH * (r_i // _C) + (r_i % _C) // _DH).astype(jnp.float32)
    s = jnp.dot(pc, S, preferred_element_type=jnp.float32)
    e = jnp.exp(s)
    rd = jax.lax.broadcasted_iota(jnp.int32, (_NS * _NH, _NH), 0)
    cd = jax.lax.broadcasted_iota(jnp.int32, (_NS * _NH, _NH), 1)
    D = (rd % _NH == cd).astype(jnp.float32)
    den = jnp.dot(e, D, preferred_element_type=jnp.float32)
    w = e * jnp.dot(1.0 / den, D.T, preferred_element_type=jnp.float32)
    wb = jnp.dot(w, S.T, preferred_element_type=jnp.float32)
    rg = jax.lax.broadcasted_iota(jnp.int32, (_NS * _C, _C), 0)
    cg = jax.lax.broadcasted_iota(jnp.int32, (_NS * _C, _C), 1)
    Gm = (rg % _C == cg).astype(jnp.float32)
    o = jnp.dot(vc * wb, Gm, preferred_element_type=jnp.float32)
    res = jnp.dot(o, outwT_ref[0], preferred_element_type=jnp.float32) + outb_ref[0]
    res = jnp.where(mx > 0.0, res, 0.0)
    ri = jax.lax.broadcasted_iota(jnp.int32, (_C, _C), 0)
    ci = jax.lax.broadcasted_iota(jnp.int32, (_C, _C), 1)
    eye = (ri == ci).astype(jnp.float32)
    resT = jax.lax.dot_general(eye, res, (((1,), (1,)), ((), ())),
                               preferred_element_type=jnp.float32)
    out_ref[0] = resT


_NC = 2
_NSUB = 16
_NW = _NC * _NSUB
_NP = 100000
_CHUNK = 20000
_TROWS = _H // _NW
_TSIZE = _TROWS * _PW
_GSIDE = _PH * _PW


def _proj_body(nw_ref, nb_ref, w_ref, b_ref, f_ref, o_ref):
    x = f_ref[0]
    mu = jnp.mean(x, axis=-1, keepdims=True)
    var = jnp.mean((x - mu) ** 2, axis=-1, keepdims=True)
    xn = (x - mu) / jnp.sqrt(var + 1e-5) * nw_ref[0] + nb_ref[0]
    o_ref[0] = jnp.dot(xn, w_ref[0], preferred_element_type=jnp.float32) + b_ref[0]


_TBSZ = 8752
_ZEND = 264200


def _sc_scatter_body(p0, p1, c0, c1, g, cbuf, tbuf, gflat, dflat, didx, rowbuf, sem):
    wid = lax.axis_index("s") * _NC + lax.axis_index("c")
    lo = jnp.where(wid == 0, 0, ((1 + _TROWS * wid) * _PW) // 8 * 8)
    hi = jnp.where(wid == _NW - 1, _ZEND, ((1 + _TROWS * (wid + 1)) * _PW) // 8 * 8)
    nown = hi - lo
    z16 = jnp.zeros((16,), jnp.float32)

    def zrow(j, _):
        for k in range(8):
            rowbuf[j, pl.ds(k * 16, 16)] = z16
        return 0

    lax.fori_loop(0, 128, zrow, 0)

    def zero_range(start, nrows):
        def zc(j, _, start=start):
            pltpu.sync_copy(rowbuf, g.at[pl.ds(start + j * 128, 128)])
            return 0

        full = nrows // 128
        lax.fori_loop(0, full, zc, 0)

        def zc8(j, _, start=start, full=full):
            pltpu.sync_copy(rowbuf.at[pl.ds(0, 8)],
                            g.at[pl.ds(start + full * 128 + j * 8, 8)])
            return 0

        lax.fori_loop(0, (nrows % 128) // 8, zc8, 0)

    zero_range(lo, nown)
    zero_range(_GSIDE + lo, nown)

    neg1 = jnp.full((16,), -1, jnp.int32)
    for s, pref, cref in ((0, p0, c0), (1, p1, c1)):
        def tinit(i, _):
            tbuf[pl.ds(i * 16, 16)] = neg1
            return 0

        lax.fori_loop(0, _TBSZ // 16, tinit, 0)
        for mc in range(_NP // _CHUNK):
            pltpu.sync_copy(cref.at[pl.ds(mc * _CHUNK, _CHUNK)], cbuf)

            def scan(i, _, pidbase=mc * _CHUNK):
                c = cbuf[pl.ds(i * 16, 16)]
                loc = c - lo
                m = (loc >= 0) & (loc < nown)
                locc = jnp.clip(loc, 0, _TBSZ - 1)
                pid = lax.iota(jnp.int32, 16) + (pidbase + i * 16)
                plsc.store_scatter(tbuf, [locc], pid, mask=m)
                return 0

            lax.fori_loop(0, _CHUNK // 16, scan, 0)

        dbase = lo + s * _GSIDE

        def comp(i, off):
            t = tbuf[pl.ds(i * 16, 16)]
            m = t >= 0
            dest = lax.iota(jnp.int32, 16) + i * 16 + dbase
            plsc.store_compressed(gflat.at[pl.ds(off, 16)], t, mask=m)
            plsc.store_compressed(dflat.at[pl.ds(off, 16)], dest, mask=m)
            return off + jnp.max(plsc.all_reduce_population_count(m))

        ncell = lax.fori_loop(0, _TBSZ // 16, comp, jnp.int32(0))
        dump = jnp.full((16,), 516 * _PW + s * _GSIDE, jnp.int32)
        zi16 = jnp.zeros((16,), jnp.int32)
        for k in range(9):
            gflat[pl.ds(ncell + k * 16, 16)] = zi16
            dflat[pl.ds(ncell + k * 16, 16)] = dump
        nch = (ncell + 127) // 128

        def mv(j, _):
            for k in range(8):
                didx[j, pl.ds(k * 16, 16)] = dflat[pl.ds(j * 128 + k * 16, 16)]
            return 0

        lax.fori_loop(0, nch, mv, 0)

        def gs(j, _, pref=pref):
            pltpu.async_copy(pref.at[gflat.at[pl.ds(j * 128, 128)]],
                             rowbuf, sem).wait()
            pltpu.async_copy(rowbuf, g.at[didx.at[j]], sem).wait()
            return 0

        lax.fori_loop(0, nch, gs, 0)


_sc_scatter = functools.partial(
    pl.kernel,
    out_type=jax.ShapeDtypeStruct((2 * _GSIDE, 128), jnp.float32),
    mesh=plsc.VectorSubcoreMesh(core_axis_name="c", subcore_axis_name="s"),
    compiler_params=pltpu.CompilerParams(needs_layout_passes=False),
    scratch_types=[
        pltpu.VMEM((_CHUNK,), jnp.int32),
        pltpu.VMEM((_TBSZ,), jnp.int32),
        pltpu.VMEM((_TBSZ + 256,), jnp.int32),
        pltpu.VMEM((_TBSZ + 256,), jnp.int32),
        pltpu.VMEM((70, 128), jnp.int32),
        pltpu.VMEM((128, 128), jnp.float32),
        pltpu.SemaphoreType.DMA,
    ],
)(_sc_scatter_body)


def _layer_norm(x, w, b):
    mu = jnp.mean(x, axis=-1, keepdims=True)
    var = jnp.mean((x - mu) ** 2, axis=-1, keepdims=True)
    return (x - mu) / jnp.sqrt(var + 1e-5) * w + b


def kernel(li_bev_feats, li_bev_coors, ra_bev_feats, ra_bev_coors,
           li_norm_w, li_norm_b, ra_norm_w, ra_norm_b,
           qkv1_qw, qkv1_qb, qkv1_kw, qkv1_kb, qkv1_vw, qkv1_vb,
           qkv2_qw, qkv2_qb, qkv2_kw, qkv2_kb, qkv2_vw, qkv2_vb,
           pos_w, pos_b,
           mha1_in_w, mha1_in_b, mha1_out_w, mha1_out_b,
           mha2_in_w, mha2_in_b, mha2_out_w, mha2_out_b):
    f32 = jnp.float32
    E = _C
    Wq1 = mha1_in_w[:E] @ qkv1_qw
    bq1 = qkv1_qb @ mha1_in_w[:E].T + mha1_in_b[:E]
    Wk1 = mha1_in_w[E:2 * E] @ qkv1_kw
    bk1 = qkv1_kb @ mha1_in_w[E:2 * E].T + mha1_in_b[E:2 * E]
    Wv1 = mha1_in_w[2 * E:] @ qkv1_vw
    bv1 = qkv1_vb @ mha1_in_w[2 * E:].T + mha1_in_b[2 * E:]
    Wq2 = mha2_in_w[:E] @ qkv2_qw
    bq2 = qkv2_qb @ mha2_in_w[:E].T + mha2_in_b[:E]
    Wk2 = mha2_in_w[E:2 * E] @ qkv2_kw
    bk2 = qkv2_kb @ mha2_in_w[E:2 * E].T + mha2_in_b[E:2 * E]
    Wv2 = mha2_in_w[2 * E:] @ qkv2_vw
    bv2 = qkv2_vb @ mha2_in_w[2 * E:].T + mha2_in_b[2 * E:]
    N = li_bev_feats.shape[1]
    zc = jnp.zeros((E,), f32)
    Wall = jnp.stack([
        jnp.concatenate([Wq1.T, Wk2.T, Wv2.T, jnp.zeros((E, E), f32)], axis=1),
        jnp.concatenate([Wq2.T, Wk1.T, Wv1.T, jnp.zeros((E, E), f32)], axis=1),
    ])
    mcol = jnp.zeros((E,), f32).at[0].set(1.0)
    ball = jnp.stack([
        jnp.concatenate([bq1, bk2, bv2, mcol]),
        jnp.concatenate([bq2, bk1, bv1, mcol]),
    ])[:, None, :]
    nrmw = jnp.stack([li_norm_w, ra_norm_w])[:, None, :]
    nrmb = jnp.stack([li_norm_b, ra_norm_b])[:, None, :]
    F = jnp.concatenate([li_bev_feats, ra_bev_feats], axis=0)
    NB = 4000
    P = pl.pallas_call(
        _proj_body,
        grid=(2, N // NB),
        in_specs=[
            pl.BlockSpec((1, 1, _C), lambda b, i: (b, 0, 0)),
            pl.BlockSpec((1, 1, _C), lambda b, i: (b, 0, 0)),
            pl.BlockSpec((1, _C, 128), lambda b, i: (b, 0, 0)),
            pl.BlockSpec((1, 1, 128), lambda b, i: (b, 0, 0)),
            pl.BlockSpec((1, NB, _C), lambda b, i: (b, i, 0)),
        ],
        out_specs=pl.BlockSpec((1, NB, 128), lambda b, i: (b, i, 0)),
        out_shape=jax.ShapeDtypeStruct((2, N, 128), f32),
    )(nrmw, nrmb, Wall, ball, F)
    cell_li = (li_bev_coors[0, :, 0] + 1) * _PW + (li_bev_coors[0, :, 1] + 1)
    cell_ra = (ra_bev_coors[0, :, 0] + 1) * _PW + (ra_bev_coors[0, :, 1] + 1)
    G = _sc_scatter(P[0], P[1], cell_li.astype(jnp.int32),
                    cell_ra.astype(jnp.int32))
    G = G.reshape(2, _PH, _PW, 128)
    sh = jnp.array(_SHIFTS, f32)
    posv = sh @ pos_w.T + pos_b
    pos2 = jnp.stack([posv @ mha1_in_w[2 * E:].T,
                      posv @ mha2_in_w[2 * E:].T], axis=0)
    bink = jnp.stack([mha1_in_b[E:2 * E], mha2_in_b[E:2 * E]])[:, None, :]
    binv = jnp.stack([mha1_in_b[2 * E:], mha2_in_b[2 * E:]])[:, None, :]
    outwT = jnp.stack([mha1_out_w.T, mha2_out_w.T])
    outb = jnp.stack([mha1_out_b, mha2_out_b])[:, None, :]

    out = pl.pallas_call(
        _attn_body,
        grid=(2, _H // _R),
        in_specs=[
            pl.BlockSpec((1, _NS, _C), lambda b, i: (b, 0, 0)),
            pl.BlockSpec((1, 1, _C), lambda b, i: (b, 0, 0)),
            pl.BlockSpec((1, 1, _C), lambda b, i: (b, 0, 0)),
            pl.BlockSpec((1, _C, _C), lambda b, i: (b, 0, 0)),
            pl.BlockSpec((1, 1, _C), lambda b, i: (b, 0, 0)),
            pl.BlockSpec((1, _R, _PW, 128), lambda b, i: (b, i, 0, 0)),
            pl.BlockSpec((1, _R, _PW, 128), lambda b, i: (b, i + 1, 0, 0)),
            pl.BlockSpec((1, _R, _PW, 128), lambda b, i: (1 - b, i, 0, 0)),
            pl.BlockSpec((1, _R, _PW, 128), lambda b, i: (1 - b, i + 1, 0, 0)),
        ],
        out_specs=pl.BlockSpec((1, _C, _R * _W), lambda b, i: (b, 0, i)),
        out_shape=jax.ShapeDtypeStruct((2, _C, _H * _W), f32),
    )(pos2, bink, binv, outwT, outb, G, G, G, G)
    out = out.reshape(2, _C, _H, _W)
    return (out[0:1], out[1:2])
